# baseline (speedup 1.0000x reference)
import jax
import jax.numpy as jnp
from jax.experimental import pallas as pl
from jax.experimental.pallas import tpu as pltpu

_B, _C, _H, _W = 8, 2, 512, 512
_BPS = 4
_STEPS = _B // _BPS


def _loss_kernel(preds_ref, labels_ref, upper_ref, out_ref, acc_ref):
    g = pl.program_id(0)

    @pl.when(g == 0)
    def _reset_total():
        acc_ref[0] = 0.0
        acc_ref[1] = 0.0

    for i in range(_BPS):
        l = labels_ref[i]
        u = upper_ref[i]
        p0 = preds_ref[i, 0]
        p1 = preds_ref[i, 1]

        neural = l == 255
        nonneural = (l == 0) & (u == 255)
        mask = neural | nonneural

        x = p0 - p1
        sp = jnp.log1p(jnp.exp(-jnp.abs(x)))
        r = jnp.maximum(jnp.where(neural, x, -x), 0.0)
        val = jnp.where(mask, r + sp, 0.0)

        s = jnp.sum(val)
        n1 = jnp.sum(neural.astype(jnp.float32))
        n2 = jnp.sum(nonneural.astype(jnp.float32))

        ok = (n1 > 0.0) & (n2 > 0.0)
        denom = n1 + n2
        contrib = s / jnp.where(denom > 0.0, denom, 1.0)
        acc_ref[0] += jnp.where(ok, contrib, 0.0)
        acc_ref[1] += jnp.where(ok, 1.0, 0.0)

    @pl.when(g == _STEPS - 1)
    def _finish():
        total = acc_ref[0]
        valid = acc_ref[1]
        out_ref[0] = jnp.where(
            valid > 0.0, total / jnp.where(valid > 0.0, valid, 1.0), 0.0
        )


def kernel(predictions, labels, upper_region):
    out = pl.pallas_call(
        _loss_kernel,
        grid=(_STEPS,),
        in_specs=[
            pl.BlockSpec((_BPS, _C, _H, _W), lambda g: (g, 0, 0, 0)),
            pl.BlockSpec((_BPS, _H, _W), lambda g: (g, 0, 0)),
            pl.BlockSpec((_BPS, _H, _W), lambda g: (g, 0, 0)),
        ],
        out_specs=pl.BlockSpec(memory_space=pltpu.SMEM),
        out_shape=jax.ShapeDtypeStruct((1,), jnp.float32),
        scratch_shapes=[pltpu.SMEM((2,), jnp.float32)],
    )(predictions, labels, upper_region)
    return out[0]


# speedup vs baseline: 1.1024x; 1.1024x over previous
import jax
import jax.numpy as jnp
from jax.experimental import pallas as pl
from jax.experimental.pallas import tpu as pltpu

_B, _C, _H, _W = 8, 2, 512, 512
_BPS = 2
_STEPS = _B // _BPS


def _loss_kernel(preds_ref, labels_ref, upper_ref, out_ref, acc_ref):
    g = pl.program_id(0)

    @pl.when(g == 0)
    def _reset_total():
        acc_ref[0] = 0.0
        acc_ref[1] = 0.0

    for i in range(_BPS):
        l = labels_ref[i]
        u = upper_ref[i]
        p0 = preds_ref[i, 0]
        p1 = preds_ref[i, 1]

        neural = l == 255
        nonneural = (l == 0) & (u == 255)
        mask = neural | nonneural

        x = p0 - p1
        sp = jnp.log1p(jnp.exp(-jnp.abs(x)))
        r = jnp.maximum(jnp.where(neural, x, -x), 0.0)
        val = jnp.where(mask, r + sp, 0.0)

        s = jnp.sum(val)
        n1 = jnp.sum(neural.astype(jnp.float32))
        n2 = jnp.sum(nonneural.astype(jnp.float32))

        ok = (n1 > 0.0) & (n2 > 0.0)
        denom = n1 + n2
        contrib = s / jnp.where(denom > 0.0, denom, 1.0)
        acc_ref[0] += jnp.where(ok, contrib, 0.0)
        acc_ref[1] += jnp.where(ok, 1.0, 0.0)

    @pl.when(g == _STEPS - 1)
    def _finish():
        total = acc_ref[0]
        valid = acc_ref[1]
        out_ref[0] = jnp.where(
            valid > 0.0, total / jnp.where(valid > 0.0, valid, 1.0), 0.0
        )


def kernel(predictions, labels, upper_region):
    out = pl.pallas_call(
        _loss_kernel,
        grid=(_STEPS,),
        in_specs=[
            pl.BlockSpec((_BPS, _C, _H, _W), lambda g: (g, 0, 0, 0)),
            pl.BlockSpec((_BPS, _H, _W), lambda g: (g, 0, 0)),
            pl.BlockSpec((_BPS, _H, _W), lambda g: (g, 0, 0)),
        ],
        out_specs=pl.BlockSpec(memory_space=pltpu.SMEM),
        out_shape=jax.ShapeDtypeStruct((1,), jnp.float32),
        scratch_shapes=[pltpu.SMEM((2,), jnp.float32)],
    )(predictions, labels, upper_region)
    return out[0]
